# baseline (speedup 1.0000x reference)
import functools
import jax
import jax.numpy as jnp
from jax import lax
from jax.experimental import pallas as pl
from jax.experimental.pallas import tpu as pltpu
from jax.experimental.pallas import tpu_sc as plsc

B = 16384
ENT = 64
NREL_PAD = 1024
NC = 2
NS = 16
NW = NC * NS
PER_W = B // NW
CHUNK = 128
NCHUNK = PER_W // CHUNK
REL_PER_S = NREL_PAD // NS


def _tec_kernel(hidx_hbm, tidx_hbm, ridx_hbm,
                head_w, head_p_w, tail_w, tail_p_w, rel_w, rel_p_w,
                out_hbm,
                rw_v, rp_v, stats_l, shared_stats, stats_v,
                hidx_v, tidx_v, ridx_v,
                hp_rows, h_rows, tp_rows, t_rows, out_buf, sem):
    cid = lax.axis_index("c")
    sid = lax.axis_index("s")
    wid = sid * NC + cid

    rbase = sid * REL_PER_S
    pltpu.sync_copy(rel_w.at[pl.ds(rbase, REL_PER_S)], rw_v)
    pltpu.sync_copy(rel_p_w.at[pl.ds(rbase, REL_PER_S)], rp_v)
    zero = jnp.zeros((16,), jnp.float32)
    for g in range(REL_PER_S // 16):
        rows = lax.iota(jnp.int32, 16) + g * 16

        def rel_body(j, carry):
            P, Q, S, R2, Sr, jj = carry
            vr = plsc.load_gather(rw_v, [rows, jj])
            vp = plsc.load_gather(rp_v, [rows, jj])
            return (P + vp * vp, Q + vp * vr, S + vp,
                    R2 + vr * vr, Sr + vr, jj + 1)

        jj0 = jnp.zeros((16,), jnp.int32)
        P, Q, S, R2, Sr, _ = lax.fori_loop(
            0, ENT, rel_body, (zero, zero, zero, zero, zero, jj0))
        sl = pl.ds(g * 16, 16)
        stats_l[0, sl] = P
        stats_l[1, sl] = Q
        stats_l[2, sl] = S
        stats_l[3, sl] = R2
        stats_l[4, sl] = Sr
    pltpu.sync_copy(stats_l, shared_stats.at[pl.ds(sid * 5, 5)])
    plsc.subcore_barrier()
    pltpu.sync_copy(shared_stats, stats_v)

    for chunk in range(NCHUNK):
        base = wid * PER_W + chunk * CHUNK
        pltpu.sync_copy(hidx_hbm.at[pl.ds(base, CHUNK)], hidx_v)
        pltpu.sync_copy(tidx_hbm.at[pl.ds(base, CHUNK)], tidx_v)
        pltpu.sync_copy(ridx_hbm.at[pl.ds(base, CHUNK)], ridx_v)
        cp1 = pltpu.async_copy(head_p_w.at[hidx_v], hp_rows, sem)
        cp2 = pltpu.async_copy(head_w.at[hidx_v], h_rows, sem)
        cp3 = pltpu.async_copy(tail_p_w.at[tidx_v], tp_rows, sem)
        cp4 = pltpu.async_copy(tail_w.at[tidx_v], t_rows, sem)
        cp1.wait()
        cp2.wait()
        cp3.wait()
        cp4.wait()
        for g in range(CHUNK // 16):
            el = lax.iota(jnp.int32, 16) + g * 16

            def ent_body(j, carry):
                aa, cc, jj = carry
                vhp = plsc.load_gather(hp_rows, [el, jj])
                vh = plsc.load_gather(h_rows, [el, jj])
                vtp = plsc.load_gather(tp_rows, [el, jj])
                vt = plsc.load_gather(t_rows, [el, jj])
                aa = aa + vhp * vh - vtp * vt
                cc = cc + vh - vt
                return aa, cc, jj + 1

            jj0 = jnp.zeros((16,), jnp.int32)
            aa, cc, _ = lax.fori_loop(0, ENT, ent_body, (zero, zero, jj0))

            rvec = ridx_v[pl.ds(g * 16, 16)]
            ri5 = lax.shift_right_logical(rvec, 6) * 5
            rj = lax.bitwise_and(rvec, 63)
            P = plsc.load_gather(stats_v, [ri5, rj])
            Q = plsc.load_gather(stats_v, [ri5 + 1, rj])
            S = plsc.load_gather(stats_v, [ri5 + 2, rj])
            R2 = plsc.load_gather(stats_v, [ri5 + 3, rj])
            Sr = plsc.load_gather(stats_v, [ri5 + 4, rj])
            score = (aa * aa * P + 2.0 * aa * Q + 2.0 * aa * cc * S
                     + R2 + 2.0 * cc * Sr + 64.0 * cc * cc)
            out_buf[pl.ds(g * 16, 16)] = score
        pltpu.sync_copy(out_buf, out_hbm.at[pl.ds(base, CHUNK)])


@jax.jit
def _transd_sc(head_indices, tail_indices, relation_indices,
               head_w, head_p_w, tail_w, tail_p_w, rel_w_pad, rel_p_w_pad):
    mesh = plsc.VectorSubcoreMesh(core_axis_name="c", subcore_axis_name="s")
    run = pl.kernel(
        _tec_kernel,
        out_type=jax.ShapeDtypeStruct((B,), jnp.float32),
        mesh=mesh,
        compiler_params=pltpu.CompilerParams(needs_layout_passes=False,
                                             use_tc_tiling_on_sc=False),
        scratch_types=[
            pltpu.VMEM((REL_PER_S, ENT), jnp.float32),
            pltpu.VMEM((REL_PER_S, ENT), jnp.float32),
            pltpu.VMEM((5, REL_PER_S), jnp.float32),
            pltpu.VMEM_SHARED((NS * 5, REL_PER_S), jnp.float32),
            pltpu.VMEM((NS * 5, REL_PER_S), jnp.float32),
            pltpu.VMEM((CHUNK,), jnp.int32),
            pltpu.VMEM((CHUNK,), jnp.int32),
            pltpu.VMEM((CHUNK,), jnp.int32),
            pltpu.VMEM((CHUNK, ENT), jnp.float32),
            pltpu.VMEM((CHUNK, ENT), jnp.float32),
            pltpu.VMEM((CHUNK, ENT), jnp.float32),
            pltpu.VMEM((CHUNK, ENT), jnp.float32),
            pltpu.VMEM((CHUNK,), jnp.float32),
            pltpu.SemaphoreType.DMA,
        ],
    )
    return run(head_indices, tail_indices, relation_indices,
               head_w, head_p_w, tail_w, tail_p_w, rel_w_pad, rel_p_w_pad)


def kernel(head_indices, tail_indices, relation_indices,
           head_w, head_p_w, tail_w, tail_p_w, rel_w, rel_p_w):
    nrel = rel_w.shape[0]
    rel_w_pad = jnp.zeros((NREL_PAD, ENT), jnp.float32).at[:nrel].set(rel_w)
    rel_p_w_pad = jnp.zeros((NREL_PAD, ENT), jnp.float32).at[:nrel].set(rel_p_w)
    return _transd_sc(head_indices.astype(jnp.int32),
                      tail_indices.astype(jnp.int32),
                      relation_indices.astype(jnp.int32),
                      head_w, head_p_w, tail_w, tail_p_w,
                      rel_w_pad, rel_p_w_pad)


# speedup vs baseline: 6.3182x; 6.3182x over previous
import functools
import jax
import jax.numpy as jnp
from jax import lax
from jax.experimental import pallas as pl
from jax.experimental.pallas import tpu as pltpu
from jax.experimental.pallas import tpu_sc as plsc

B = 16384
ENT = 64
NNODES = 1000000
NREL = 1000
NREL_PAD = 1024
NC = 2
NS = 16
NW = NC * NS
PER_W = B // NW
CHUNK = 128
NCHUNK = PER_W // CHUNK
RW = 8192
RSTEPS = (NNODES + RW - 1) // RW


def _reduce_body(hw, hpw, tw, tpw, dh, sh, dt, st):
    c = pl.program_id(0)
    col = jax.lax.broadcasted_iota(jnp.int32, (1, RW), 1) + c * RW
    m = (col < NNODES).astype(jnp.float32)
    h = hw[...] * m
    hp = hpw[...] * m
    t = tw[...] * m
    tp = tpw[...] * m
    dh[...] = jnp.sum(hp * h, axis=0)
    sh[...] = jnp.sum(h, axis=0)
    dt[...] = jnp.sum(tp * t, axis=0)
    st[...] = jnp.sum(t, axis=0)


def _stats_body(rw, rpw, P, Q, S, R2, Sr):
    col = jax.lax.broadcasted_iota(jnp.int32, (1, NREL_PAD), 1)
    m = (col < NREL).astype(jnp.float32)
    r = rw[...] * m
    rp = rpw[...] * m
    P[...] = jnp.sum(rp * rp, axis=0)
    Q[...] = jnp.sum(rp * r, axis=0)
    S[...] = jnp.sum(rp, axis=0)
    R2[...] = jnp.sum(r * r, axis=0)
    Sr[...] = jnp.sum(r, axis=0)


def _score_body(hidx_hbm, tidx_hbm, ridx_hbm, dh_hbm, sh_hbm, dt_hbm, st_hbm,
                P_hbm, Q_hbm, S_hbm, R2_hbm, Sr_hbm, out_hbm,
                P_v, Q_v, S_v, R2_v, Sr_v, hidx_v, tidx_v, ridx_v,
                dh_v, sh_v, dt_v, st_v, out_buf, sem):
    cid = lax.axis_index("c")
    sid = lax.axis_index("s")
    wid = sid * NC + cid
    pltpu.sync_copy(P_hbm, P_v)
    pltpu.sync_copy(Q_hbm, Q_v)
    pltpu.sync_copy(S_hbm, S_v)
    pltpu.sync_copy(R2_hbm, R2_v)
    pltpu.sync_copy(Sr_hbm, Sr_v)
    for chunk in range(NCHUNK):
        base = wid * PER_W + chunk * CHUNK
        pltpu.sync_copy(hidx_hbm.at[pl.ds(base, CHUNK)], hidx_v)
        pltpu.sync_copy(tidx_hbm.at[pl.ds(base, CHUNK)], tidx_v)
        pltpu.sync_copy(ridx_hbm.at[pl.ds(base, CHUNK)], ridx_v)
        cp1 = pltpu.async_copy(dh_hbm.at[hidx_v], dh_v, sem)
        cp2 = pltpu.async_copy(sh_hbm.at[hidx_v], sh_v, sem)
        cp3 = pltpu.async_copy(dt_hbm.at[tidx_v], dt_v, sem)
        cp4 = pltpu.async_copy(st_hbm.at[tidx_v], st_v, sem)
        cp1.wait()
        cp2.wait()
        cp3.wait()
        cp4.wait()
        for g in range(CHUNK // 16):
            sl = pl.ds(g * 16, 16)
            aa = dh_v[sl] - dt_v[sl]
            cc = sh_v[sl] - st_v[sl]
            rvec = ridx_v[sl]
            Pv = plsc.load_gather(P_v, [rvec])
            Qv = plsc.load_gather(Q_v, [rvec])
            Sv = plsc.load_gather(S_v, [rvec])
            R2v = plsc.load_gather(R2_v, [rvec])
            Srv = plsc.load_gather(Sr_v, [rvec])
            score = (aa * aa * Pv + 2.0 * aa * Qv + 2.0 * aa * cc * Sv
                     + R2v + 2.0 * cc * Srv + 64.0 * cc * cc)
            out_buf[sl] = score
        pltpu.sync_copy(out_buf, out_hbm.at[pl.ds(base, CHUNK)])


@jax.jit
def _transd(head_indices, tail_indices, relation_indices,
            head_w, head_p_w, tail_w, tail_p_w, rel_w, rel_p_w):
    hw_t = head_w.T
    hpw_t = head_p_w.T
    tw_t = tail_w.T
    tpw_t = tail_p_w.T

    vec = jax.ShapeDtypeStruct((NNODES,), jnp.float32)
    dh, sh, dt, st = pl.pallas_call(
        _reduce_body,
        grid=(RSTEPS,),
        in_specs=[pl.BlockSpec((ENT, RW), lambda c: (0, c))] * 4,
        out_specs=[pl.BlockSpec((RW,), lambda c: (c,))] * 4,
        out_shape=[vec] * 4,
    )(hw_t, hpw_t, tw_t, tpw_t)

    rvec = jax.ShapeDtypeStruct((NREL_PAD,), jnp.float32)
    P, Q, S, R2, Sr = pl.pallas_call(
        _stats_body,
        grid=(1,),
        in_specs=[pl.BlockSpec((ENT, NREL_PAD), lambda c: (0, 0))] * 2,
        out_specs=[pl.BlockSpec((NREL_PAD,), lambda c: (0,))] * 5,
        out_shape=[rvec] * 5,
    )(rel_w.T, rel_p_w.T)

    mesh = plsc.VectorSubcoreMesh(core_axis_name="c", subcore_axis_name="s")
    run = pl.kernel(
        _score_body,
        out_type=jax.ShapeDtypeStruct((B,), jnp.float32),
        mesh=mesh,
        compiler_params=pltpu.CompilerParams(needs_layout_passes=False),
        scratch_types=[
            pltpu.VMEM((NREL_PAD,), jnp.float32),
            pltpu.VMEM((NREL_PAD,), jnp.float32),
            pltpu.VMEM((NREL_PAD,), jnp.float32),
            pltpu.VMEM((NREL_PAD,), jnp.float32),
            pltpu.VMEM((NREL_PAD,), jnp.float32),
            pltpu.VMEM((CHUNK,), jnp.int32),
            pltpu.VMEM((CHUNK,), jnp.int32),
            pltpu.VMEM((CHUNK,), jnp.int32),
            pltpu.VMEM((CHUNK,), jnp.float32),
            pltpu.VMEM((CHUNK,), jnp.float32),
            pltpu.VMEM((CHUNK,), jnp.float32),
            pltpu.VMEM((CHUNK,), jnp.float32),
            pltpu.VMEM((CHUNK,), jnp.float32),
            pltpu.SemaphoreType.DMA,
        ],
    )
    return run(head_indices, tail_indices, relation_indices,
               dh, sh, dt, st, P, Q, S, R2, Sr)


def kernel(head_indices, tail_indices, relation_indices,
           head_w, head_p_w, tail_w, tail_p_w, rel_w, rel_p_w):
    return _transd(head_indices.astype(jnp.int32),
                   tail_indices.astype(jnp.int32),
                   relation_indices.astype(jnp.int32),
                   head_w, head_p_w, tail_w, tail_p_w, rel_w, rel_p_w)


# speedup vs baseline: 6.3880x; 1.0110x over previous
import functools
import jax
import jax.numpy as jnp
from jax import lax
from jax.experimental import pallas as pl
from jax.experimental.pallas import tpu as pltpu
from jax.experimental.pallas import tpu_sc as plsc

B = 16384
ENT = 64
NNODES = 1000000
NREL = 1000
NREL_PAD = 1024
NC = 2
NS = 16
NW = NC * NS
PER_W = B // NW
CHUNK = 128
NCHUNK = PER_W // CHUNK
RW = 16384
RSTEPS = (NNODES + RW - 1) // RW


def _reduce_body(hw, hpw, tw, tpw, dh, sh, dt, st):
    h = hw[...]
    t = tw[...]
    dh[...] = jnp.sum(hpw[...] * h, axis=0)
    sh[...] = jnp.sum(h, axis=0)
    dt[...] = jnp.sum(tpw[...] * t, axis=0)
    st[...] = jnp.sum(t, axis=0)


def _stats_body(rw, rpw, P, Q, S, R2, Sr):
    col = jax.lax.broadcasted_iota(jnp.int32, (1, NREL_PAD), 1)
    m = (col < NREL).astype(jnp.float32)
    r = rw[...] * m
    rp = rpw[...] * m
    P[...] = jnp.sum(rp * rp, axis=0)
    Q[...] = jnp.sum(rp * r, axis=0)
    S[...] = jnp.sum(rp, axis=0)
    R2[...] = jnp.sum(r * r, axis=0)
    Sr[...] = jnp.sum(r, axis=0)


def _score_body(hidx_hbm, tidx_hbm, ridx_hbm, dh_hbm, sh_hbm, dt_hbm, st_hbm,
                P_hbm, Q_hbm, S_hbm, R2_hbm, Sr_hbm, out_hbm,
                P_v, Q_v, S_v, R2_v, Sr_v, hidx_v, tidx_v, ridx_v,
                dh_v, sh_v, dt_v, st_v, out_buf, sem):
    cid = lax.axis_index("c")
    sid = lax.axis_index("s")
    wid = sid * NC + cid
    pltpu.sync_copy(P_hbm, P_v)
    pltpu.sync_copy(Q_hbm, Q_v)
    pltpu.sync_copy(S_hbm, S_v)
    pltpu.sync_copy(R2_hbm, R2_v)
    pltpu.sync_copy(Sr_hbm, Sr_v)
    for chunk in range(NCHUNK):
        base = wid * PER_W + chunk * CHUNK
        pltpu.sync_copy(hidx_hbm.at[pl.ds(base, CHUNK)], hidx_v)
        pltpu.sync_copy(tidx_hbm.at[pl.ds(base, CHUNK)], tidx_v)
        pltpu.sync_copy(ridx_hbm.at[pl.ds(base, CHUNK)], ridx_v)
        cp1 = pltpu.async_copy(dh_hbm.at[hidx_v], dh_v, sem)
        cp2 = pltpu.async_copy(sh_hbm.at[hidx_v], sh_v, sem)
        cp3 = pltpu.async_copy(dt_hbm.at[tidx_v], dt_v, sem)
        cp4 = pltpu.async_copy(st_hbm.at[tidx_v], st_v, sem)
        cp1.wait()
        cp2.wait()
        cp3.wait()
        cp4.wait()
        for g in range(CHUNK // 16):
            sl = pl.ds(g * 16, 16)
            aa = dh_v[sl] - dt_v[sl]
            cc = sh_v[sl] - st_v[sl]
            rvec = ridx_v[sl]
            Pv = plsc.load_gather(P_v, [rvec])
            Qv = plsc.load_gather(Q_v, [rvec])
            Sv = plsc.load_gather(S_v, [rvec])
            R2v = plsc.load_gather(R2_v, [rvec])
            Srv = plsc.load_gather(Sr_v, [rvec])
            score = (aa * aa * Pv + 2.0 * aa * Qv + 2.0 * aa * cc * Sv
                     + R2v + 2.0 * cc * Srv + 64.0 * cc * cc)
            out_buf[sl] = score
        pltpu.sync_copy(out_buf, out_hbm.at[pl.ds(base, CHUNK)])


@jax.jit
def _transd(head_indices, tail_indices, relation_indices,
            head_w, head_p_w, tail_w, tail_p_w, rel_w, rel_p_w):
    hw_t = head_w.T
    hpw_t = head_p_w.T
    tw_t = tail_w.T
    tpw_t = tail_p_w.T

    vec = jax.ShapeDtypeStruct((NNODES,), jnp.float32)
    dh, sh, dt, st = pl.pallas_call(
        _reduce_body,
        grid=(RSTEPS,),
        in_specs=[pl.BlockSpec((ENT, RW), lambda c: (0, c))] * 4,
        out_specs=[pl.BlockSpec((RW,), lambda c: (c,))] * 4,
        out_shape=[vec] * 4,
    )(hw_t, hpw_t, tw_t, tpw_t)

    rvec = jax.ShapeDtypeStruct((NREL_PAD,), jnp.float32)
    P, Q, S, R2, Sr = pl.pallas_call(
        _stats_body,
        grid=(1,),
        in_specs=[pl.BlockSpec((ENT, NREL_PAD), lambda c: (0, 0))] * 2,
        out_specs=[pl.BlockSpec((NREL_PAD,), lambda c: (0,))] * 5,
        out_shape=[rvec] * 5,
    )(rel_w.T, rel_p_w.T)

    mesh = plsc.VectorSubcoreMesh(core_axis_name="c", subcore_axis_name="s")
    run = pl.kernel(
        _score_body,
        out_type=jax.ShapeDtypeStruct((B,), jnp.float32),
        mesh=mesh,
        compiler_params=pltpu.CompilerParams(needs_layout_passes=False),
        scratch_types=[
            pltpu.VMEM((NREL_PAD,), jnp.float32),
            pltpu.VMEM((NREL_PAD,), jnp.float32),
            pltpu.VMEM((NREL_PAD,), jnp.float32),
            pltpu.VMEM((NREL_PAD,), jnp.float32),
            pltpu.VMEM((NREL_PAD,), jnp.float32),
            pltpu.VMEM((CHUNK,), jnp.int32),
            pltpu.VMEM((CHUNK,), jnp.int32),
            pltpu.VMEM((CHUNK,), jnp.int32),
            pltpu.VMEM((CHUNK,), jnp.float32),
            pltpu.VMEM((CHUNK,), jnp.float32),
            pltpu.VMEM((CHUNK,), jnp.float32),
            pltpu.VMEM((CHUNK,), jnp.float32),
            pltpu.VMEM((CHUNK,), jnp.float32),
            pltpu.SemaphoreType.DMA,
        ],
    )
    return run(head_indices, tail_indices, relation_indices,
               dh, sh, dt, st, P, Q, S, R2, Sr)


def kernel(head_indices, tail_indices, relation_indices,
           head_w, head_p_w, tail_w, tail_p_w, rel_w, rel_p_w):
    return _transd(head_indices.astype(jnp.int32),
                   tail_indices.astype(jnp.int32),
                   relation_indices.astype(jnp.int32),
                   head_w, head_p_w, tail_w, tail_p_w, rel_w, rel_p_w)
